# baseline (speedup 1.0000x reference)
import jax
import jax.numpy as jnp
from jax.experimental import pallas as pl
from jax.experimental.pallas import tpu as pltpu

_TB = 256
_NC = 24


def _main(d_ref, pk_ref,
          e_bt, wt_bt, b_bt, e_st, wt_st, b_st, e_cj, wt_cj, b_cj,
          wf0_ref, bf0_ref, w1_ref, b1_ref, w2_ref, b2_ref,
          o_ref, ws_ref):
    @pl.when(pl.program_id(0) == 0)
    def _build_tables():
        f0 = wf0_ref[...]
        u_bt = ((e_bt[...] @ wt_bt[...]) + b_bt[...]) @ f0 + bf0_ref[...]
        u_st = ((e_st[...] @ wt_st[...]) + b_st[...]) @ f0
        u_cj = ((e_cj[...] @ wt_cj[...]) + b_cj[...]) @ f0
        d0 = f0.shape[0] - 1
        sd_row = f0[d0:d0 + 1, :]
        pad = jnp.zeros((_NC - 17, f0.shape[1]), jnp.float32)
        ws_ref[...] = jnp.concatenate([u_bt, u_st, u_cj, sd_row, pad], axis=0)

    pk = pk_ref[...]
    sd = pk[0]
    bt = pk[1]
    st = pk[2]
    cj = pk[3]
    nsub = sd.shape[0]
    iic = jax.lax.broadcasted_iota(
        jnp.int32, (_NC, sd.shape[1]), 0).astype(jnp.float32)
    chunks = []
    for s in range(nsub):
        ohc = ((iic == jnp.broadcast_to(bt[s:s + 1, :], iic.shape))
               | (iic == jnp.broadcast_to(st[s:s + 1, :], iic.shape) + 8.0)
               | (iic == jnp.broadcast_to(cj[s:s + 1, :], iic.shape) + 14.0))
        sdb = jnp.broadcast_to(sd[s:s + 1, :], iic.shape)
        chunks.append(jnp.where(iic == 16.0, sdb, ohc.astype(jnp.float32)))
    gt = jnp.concatenate(chunks, axis=1)
    g = jnp.transpose(gt).astype(jnp.bfloat16)
    d = d_ref.shape[1]
    w0 = wf0_ref[0:d, :]
    h = jnp.dot(d_ref[...].astype(jnp.bfloat16), w0.astype(jnp.bfloat16),
                preferred_element_type=jnp.float32)
    h = h + jnp.dot(g, ws_ref[...].astype(jnp.bfloat16),
                    preferred_element_type=jnp.float32)
    h = jnp.maximum(h, 0.0)
    h = jnp.dot(h.astype(jnp.bfloat16), w1_ref[...].astype(jnp.bfloat16),
                preferred_element_type=jnp.float32)
    h = jnp.maximum(h + b1_ref[...], 0.0)
    o = jnp.sum(h * w2_ref[...], axis=1, keepdims=True) + b2_ref[...]
    o_ref[...] = o


def kernel(atom, degree, hybridization, chirality, formal_charge, partial_charge, pos, dist, spatial_distance, bond_type, stereo, conjugated, emb_atom, emb_degree, emb_hybridization, emb_chirality, emb_bond_type, emb_stereo, emb_conjugated, W_charge, b_charge, W_pos, b_pos, Wt_bond_type, bt_bond_type, Wt_stereo, bt_stereo, Wt_conjugated, bt_conjugated, Wf0, bf0, Wf1, bf1, Wf2, bf2):
    b, n = spatial_distance.shape[0], spatial_distance.shape[1]
    d = dist.shape[-1]
    dm = Wf0.shape[0]
    h1 = Wf0.shape[1]
    h2 = Wf1.shape[1]
    m = b * n * n
    tm = min(_TB * n, m)
    lanes = 128
    msub = m // lanes
    nsub = tm // lanes

    dist2 = dist.reshape(m, d)
    packed = jnp.stack([
        spatial_distance.reshape(msub, lanes),
        bond_type.astype(jnp.float32).reshape(msub, lanes),
        stereo.astype(jnp.float32).reshape(msub, lanes),
        conjugated.astype(jnp.float32).reshape(msub, lanes),
    ])

    grid = (m // tm,)

    def full2(a_, b_):
        return pl.BlockSpec((a_, b_), lambda i: (0, 0))

    out2 = pl.pallas_call(
        _main,
        grid=grid,
        in_specs=[
            pl.BlockSpec((tm, d), lambda i: (i, 0)),
            pl.BlockSpec((4, nsub, lanes), lambda i: (0, i, 0)),
            full2(*emb_bond_type.shape), full2(*Wt_bond_type.shape),
            full2(1, dm),
            full2(*emb_stereo.shape), full2(*Wt_stereo.shape),
            full2(1, dm),
            full2(*emb_conjugated.shape), full2(*Wt_conjugated.shape),
            full2(1, dm),
            full2(*Wf0.shape), full2(1, h1),
            full2(*Wf1.shape), full2(1, h2),
            full2(1, h2), full2(1, 1),
        ],
        out_specs=pl.BlockSpec((tm, 1), lambda i: (i, 0)),
        out_shape=jax.ShapeDtypeStruct((m, 1), jnp.float32),
        scratch_shapes=[pltpu.VMEM((_NC, h1), jnp.float32)],
        compiler_params=pltpu.CompilerParams(
            dimension_semantics=("arbitrary",)),
    )(dist2, packed,
      emb_bond_type, Wt_bond_type, bt_bond_type.reshape(1, -1),
      emb_stereo, Wt_stereo, bt_stereo.reshape(1, -1),
      emb_conjugated, Wt_conjugated, bt_conjugated.reshape(1, -1),
      Wf0, bf0.reshape(1, -1), Wf1, bf1.reshape(1, -1),
      Wf2.reshape(1, -1), bf2.reshape(1, -1))

    return out2.reshape(b, n, n)
